# Initial kernel scaffold; baseline (speedup 1.0000x reference)
import jax
import jax.numpy as jnp
from jax.experimental import pallas as pl


def kernel(group, weights, ray_indices, num_rays):
    raise NotImplementedError("write your pallas kernel here")


# speedup vs baseline: 2.1374x; 2.1374x over previous
import functools

import jax
import jax.numpy as jnp
from jax import lax
from jax.experimental import pallas as pl
from jax.experimental.pallas import tpu as pltpu
from jax.experimental.pallas import tpu_sc as plsc

N = 320000
D = 128
R_OUT = 10000
NC, NS = 2, 16
NW = NC * NS
R_PER_W = (R_OUT + NW - 1) // NW
R_PAD = NW * R_PER_W
C = 512
ACC_W = R_PER_W * D
LANES = 16


def _sc_body(g_hbm, w_hbm, i_hbm, b_hbm, out_hbm, gbuf, wbuf, ibuf, bbuf, acc):
    wid = lax.axis_index("s") * NC + lax.axis_index("c")
    r_base = wid * R_PER_W

    zeros = jnp.zeros((LANES,), jnp.float32)

    def zbody(k, _):
        acc[pl.ds(k * LANES, LANES)] = zeros
        return _

    lax.fori_loop(0, ACC_W // LANES, zbody, None)

    pltpu.sync_copy(b_hbm, bbuf)
    bv = bbuf[pl.ds(wid * LANES, LANES)]
    s0 = bv[0]
    s1 = bv[1]
    s0a = (s0 // LANES) * LANES
    nchunks = (s1 - s0a + C - 1) // C
    lanes = lax.iota(jnp.int32, LANES)

    def chunk_body(c, _):
        nom = s0a + c * C
        cs = jnp.minimum(nom, N - C)
        vlo = jnp.maximum(s0, nom)
        pltpu.sync_copy(g_hbm.at[pl.ds(cs * D, C * D)], gbuf)
        pltpu.sync_copy(w_hbm.at[pl.ds(cs, C)], wbuf)
        pltpu.sync_copy(i_hbm.at[pl.ds(cs, C)], ibuf)

        def sbody(i16, _):
            base = i16 * LANES
            widx = ibuf[pl.ds(base, LANES)]
            wv = wbuf[pl.ds(base, LANES)]
            gav = (cs + base) + lanes
            vmask = jnp.logical_and(gav >= vlo, gav < s1)
            wv = jnp.where(vmask, wv, 0.0)
            obasev = jnp.clip(widx - r_base, 0, R_PER_W - 1) * D
            for l in range(LANES):
                wgt = wv[l]
                obase = obasev[l]
                ibase = (base + l) * D
                for j in range(D // LANES):
                    row = gbuf[pl.ds(ibase + j * LANES, LANES)]
                    plsc.addupdate(acc.at[pl.ds(obase + j * LANES, LANES)],
                                   row * wgt)
            return _

        lax.fori_loop(0, C // LANES, sbody, None)
        return _

    lax.fori_loop(0, nchunks, chunk_body, None)

    pltpu.sync_copy(acc, out_hbm.at[wid])


@jax.jit
def _sc_segment_sum(g_flat, w_flat, idx, bounds):
    mesh = plsc.VectorSubcoreMesh(core_axis_name="c", subcore_axis_name="s",
                                  num_cores=NC, num_subcores=NS)
    f = pl.kernel(
        _sc_body,
        out_type=jax.ShapeDtypeStruct((NW, ACC_W), jnp.float32),
        mesh=mesh,
        scratch_types=[
            pltpu.VMEM((C * D,), jnp.float32),
            pltpu.VMEM((C,), jnp.float32),
            pltpu.VMEM((C,), jnp.int32),
            pltpu.VMEM((NW * LANES,), jnp.int32),
            pltpu.VMEM((ACC_W,), jnp.float32),
        ],
    )
    return f(g_flat, w_flat, idx, bounds)


def kernel(group, weights, ray_indices, num_rays):
    del num_rays
    idx = ray_indices.astype(jnp.int32)
    qs = jnp.arange(NW + 1, dtype=jnp.int32) * R_PER_W
    b = jnp.searchsorted(idx, qs, side="left").astype(jnp.int32)
    bounds = jnp.pad(jnp.stack([b[:-1], b[1:]], axis=1),
                     ((0, 0), (0, LANES - 2))).reshape(NW * LANES)
    g_flat = group.reshape(N * D)
    w_flat = weights.reshape(N)
    res = _sc_segment_sum(g_flat, w_flat, idx, bounds)
    return res.reshape(R_PAD, D)[:R_OUT]


# speedup vs baseline: 2.4910x; 1.1654x over previous
import functools

import jax
import jax.numpy as jnp
from jax import lax
from jax.experimental import pallas as pl
from jax.experimental.pallas import tpu as pltpu
from jax.experimental.pallas import tpu_sc as plsc

N = 320000
D = 128
R_OUT = 10000
NC, NS = 2, 16
NW = NC * NS
R_PER_W = (R_OUT + NW - 1) // NW
R_PAD = NW * R_PER_W
C = 256
ACC_W = R_PER_W * D
LANES = 16


def _sc_body(g_hbm, w_hbm, i_hbm, b_hbm, out_hbm,
             g0, g1, w0, w1, i0, i1, bbuf, acc, sem0, sem1):
    wid = lax.axis_index("s") * NC + lax.axis_index("c")
    r_base = wid * R_PER_W

    zeros = jnp.zeros((LANES,), jnp.float32)

    def zbody(k, _):
        acc[pl.ds(k * LANES, LANES)] = zeros
        return _

    lax.fori_loop(0, ACC_W // LANES, zbody, None)

    pltpu.sync_copy(b_hbm, bbuf)
    bv = bbuf[pl.ds(wid * LANES, LANES)]
    s0 = bv[0]
    s1 = bv[1]
    s0a = (s0 // LANES) * LANES
    nchunks = (s1 - s0a + C - 1) // C
    lanes = lax.iota(jnp.int32, LANES)

    def start_chunk(c, gb, wb, ib, sem):
        cs = jnp.minimum(s0a + c * C, N - C)
        pltpu.make_async_copy(g_hbm.at[pl.ds(cs * D, C * D)], gb, sem).start()
        pltpu.make_async_copy(w_hbm.at[pl.ds(cs, C)], wb, sem).start()
        pltpu.make_async_copy(i_hbm.at[pl.ds(cs, C)], ib, sem).start()

    def wait_chunk(gb, wb, ib, sem):
        pltpu.make_async_copy(g_hbm.at[pl.ds(0, C * D)], gb, sem).wait()
        pltpu.make_async_copy(w_hbm.at[pl.ds(0, C)], wb, sem).wait()
        pltpu.make_async_copy(i_hbm.at[pl.ds(0, C)], ib, sem).wait()

    def compute(c, gb, wb, ib):
        nom = s0a + c * C
        cs = jnp.minimum(nom, N - C)
        vlo = jnp.maximum(s0, nom)

        def sbody(i16, _):
            base = i16 * LANES
            widx = ib[pl.ds(base, LANES)]
            wv = wb[pl.ds(base, LANES)]
            gav = (cs + base) + lanes
            vmask = jnp.logical_and(gav >= vlo, gav < s1)
            wv = jnp.where(vmask, wv, 0.0)
            obasev = jnp.clip(widx - r_base, 0, R_PER_W - 1) * D
            for l in range(LANES):
                wgt = wv[l]
                obase = obasev[l]
                ibase = (base + l) * D
                for j in range(D // LANES):
                    row = gb[pl.ds(ibase + j * LANES, LANES)]
                    plsc.addupdate(acc.at[pl.ds(obase + j * LANES, LANES)],
                                   row * wgt)
            return _

        lax.fori_loop(0, C // LANES, sbody, None)

    start_chunk(0, g0, w0, i0, sem0)

    def outer(co, _):
        c = 2 * co
        start_chunk(c + 1, g1, w1, i1, sem1)
        wait_chunk(g0, w0, i0, sem0)
        compute(c, g0, w0, i0)
        start_chunk(c + 2, g0, w0, i0, sem0)
        wait_chunk(g1, w1, i1, sem1)
        compute(c + 1, g1, w1, i1)
        return _

    lax.fori_loop(0, (nchunks + 1) // 2, outer, None)
    wait_chunk(g0, w0, i0, sem0)

    pltpu.sync_copy(acc, out_hbm.at[wid])


@jax.jit
def _sc_segment_sum(g_flat, w_flat, idx, bounds):
    mesh = plsc.VectorSubcoreMesh(core_axis_name="c", subcore_axis_name="s",
                                  num_cores=NC, num_subcores=NS)
    f = pl.kernel(
        _sc_body,
        out_type=jax.ShapeDtypeStruct((NW, ACC_W), jnp.float32),
        mesh=mesh,
        scratch_types=[
            pltpu.VMEM((C * D,), jnp.float32),
            pltpu.VMEM((C * D,), jnp.float32),
            pltpu.VMEM((C,), jnp.float32),
            pltpu.VMEM((C,), jnp.float32),
            pltpu.VMEM((C,), jnp.int32),
            pltpu.VMEM((C,), jnp.int32),
            pltpu.VMEM((NW * LANES,), jnp.int32),
            pltpu.VMEM((ACC_W,), jnp.float32),
            pltpu.SemaphoreType.DMA,
            pltpu.SemaphoreType.DMA,
        ],
    )
    return f(g_flat, w_flat, idx, bounds)


def kernel(group, weights, ray_indices, num_rays):
    del num_rays
    idx = ray_indices.astype(jnp.int32)
    qs = jnp.arange(NW + 1, dtype=jnp.int32) * R_PER_W
    b = jnp.searchsorted(idx, qs, side="left").astype(jnp.int32)
    bounds = jnp.pad(jnp.stack([b[:-1], b[1:]], axis=1),
                     ((0, 0), (0, LANES - 2))).reshape(NW * LANES)
    g_flat = group.reshape(N * D)
    w_flat = weights.reshape(N)
    res = _sc_segment_sum(g_flat, w_flat, idx, bounds)
    return res.reshape(R_PAD, D)[:R_OUT]


# speedup vs baseline: 5.4568x; 2.1906x over previous
import functools

import jax
import jax.numpy as jnp
from jax import lax
from jax.experimental import pallas as pl
from jax.experimental.pallas import tpu as pltpu
from jax.experimental.pallas import tpu_sc as plsc

N = 320000
D = 128
R_OUT = 10000
NC, NS = 2, 16
NW = NC * NS
R_PER_W = (R_OUT + NW - 1) // NW
R_PAD = NW * R_PER_W
C = 256
ACC_W = R_PER_W * D
LANES = 16


def _sc_body(g_hbm, w_hbm, i_hbm, b_hbm, out_hbm,
             g0, g1, w0, w1, i0, i1, bbuf, acc, sem0, sem1):
    wid = lax.axis_index("s") * NC + lax.axis_index("c")
    r_base = wid * R_PER_W

    zeros = jnp.zeros((LANES,), jnp.float32)

    def zbody(k, _):
        acc[pl.ds(k * LANES, LANES)] = zeros
        return _

    lax.fori_loop(0, ACC_W // LANES, zbody, None)

    pltpu.sync_copy(b_hbm, bbuf)
    bv = bbuf[pl.ds(wid * LANES, LANES)]
    s0 = bv[0]
    s1 = bv[1]
    s0a = (s0 // LANES) * LANES
    nchunks = (s1 - s0a + C - 1) // C
    lanes = lax.iota(jnp.int32, LANES)

    def start_chunk(c, gb, wb, ib, sem):
        cs = jnp.minimum(s0a + c * C, N - C)
        pltpu.make_async_copy(g_hbm.at[pl.ds(cs * D, C * D)], gb, sem).start()
        pltpu.make_async_copy(w_hbm.at[pl.ds(cs, C)], wb, sem).start()
        pltpu.make_async_copy(i_hbm.at[pl.ds(cs, C)], ib, sem).start()

    def wait_chunk(gb, wb, ib, sem):
        pltpu.make_async_copy(g_hbm.at[pl.ds(0, C * D)], gb, sem).wait()
        pltpu.make_async_copy(w_hbm.at[pl.ds(0, C)], wb, sem).wait()
        pltpu.make_async_copy(i_hbm.at[pl.ds(0, C)], ib, sem).wait()

    def flush(prev, avs):
        for j in range(D // LANES):
            plsc.addupdate(acc.at[pl.ds(prev + j * LANES, LANES)], avs[j])

    def compute(c, gb, wb, ib, run):
        nom = s0a + c * C
        cs = jnp.minimum(nom, N - C)
        vlo = jnp.maximum(s0, nom)

        def sbody(i16, run):
            base = i16 * LANES
            widx = ib[pl.ds(base, LANES)]
            wv = wb[pl.ds(base, LANES)]
            gav = (cs + base) + lanes
            vmask = jnp.logical_and(gav >= vlo, gav < s1)
            wv = jnp.where(vmask, wv, 0.0)
            obasev = jnp.clip(widx - r_base, 0, R_PER_W - 1) * D
            prev, avs = run
            for l in range(LANES):
                wgt = wv[l]
                obase = obasev[l]
                ibase = (base + l) * D

                def new_run(prev=prev, avs=avs):
                    flush(prev, avs)
                    return (jnp.zeros((LANES,), jnp.float32),) * (D // LANES)

                avs = lax.cond(obase != prev, new_run, lambda avs=avs: avs)
                avs = tuple(
                    avs[j] + gb[pl.ds(ibase + j * LANES, LANES)] * wgt
                    for j in range(D // LANES))
                prev = obase
            return (prev, avs)

        return lax.fori_loop(0, C // LANES, sbody, run)

    start_chunk(0, g0, w0, i0, sem0)
    run0 = (jnp.int32(R_PER_W * D),
            (jnp.zeros((LANES,), jnp.float32),) * (D // LANES))

    def outer(co, run):
        c = 2 * co
        start_chunk(c + 1, g1, w1, i1, sem1)
        wait_chunk(g0, w0, i0, sem0)
        run = compute(c, g0, w0, i0, run)
        start_chunk(c + 2, g0, w0, i0, sem0)
        wait_chunk(g1, w1, i1, sem1)
        run = compute(c + 1, g1, w1, i1, run)
        return run

    run = lax.fori_loop(0, (nchunks + 1) // 2, outer, run0)
    flush(run[0], run[1])
    wait_chunk(g0, w0, i0, sem0)

    pltpu.sync_copy(acc.at[pl.ds(0, ACC_W)], out_hbm.at[wid])


@jax.jit
def _sc_segment_sum(g_flat, w_flat, idx, bounds):
    mesh = plsc.VectorSubcoreMesh(core_axis_name="c", subcore_axis_name="s",
                                  num_cores=NC, num_subcores=NS)
    f = pl.kernel(
        _sc_body,
        out_type=jax.ShapeDtypeStruct((NW, ACC_W), jnp.float32),
        mesh=mesh,
        scratch_types=[
            pltpu.VMEM((C * D,), jnp.float32),
            pltpu.VMEM((C * D,), jnp.float32),
            pltpu.VMEM((C,), jnp.float32),
            pltpu.VMEM((C,), jnp.float32),
            pltpu.VMEM((C,), jnp.int32),
            pltpu.VMEM((C,), jnp.int32),
            pltpu.VMEM((NW * LANES,), jnp.int32),
            pltpu.VMEM((ACC_W + D,), jnp.float32),
            pltpu.SemaphoreType.DMA,
            pltpu.SemaphoreType.DMA,
        ],
    )
    return f(g_flat, w_flat, idx, bounds)


def kernel(group, weights, ray_indices, num_rays):
    del num_rays
    idx = ray_indices.astype(jnp.int32)
    qs = jnp.arange(NW + 1, dtype=jnp.int32) * R_PER_W
    b = jnp.searchsorted(idx, qs, side="left").astype(jnp.int32)
    bounds = jnp.pad(jnp.stack([b[:-1], b[1:]], axis=1),
                     ((0, 0), (0, LANES - 2))).reshape(NW * LANES)
    g_flat = group.reshape(N * D)
    w_flat = weights.reshape(N)
    res = _sc_segment_sum(g_flat, w_flat, idx, bounds)
    return res.reshape(R_PAD, D)[:R_OUT]


# speedup vs baseline: 6.4524x; 1.1825x over previous
import functools

import jax
import jax.numpy as jnp
from jax import lax
from jax.experimental import pallas as pl
from jax.experimental.pallas import tpu as pltpu
from jax.experimental.pallas import tpu_sc as plsc

N = 320000
D = 128
R_OUT = 10000
NC, NS = 2, 16
NW = NC * NS
R_PER_W = (R_OUT + NW - 1) // NW
R_PAD = NW * R_PER_W
C = 256
ACC_W = R_PER_W * D
LANES = 16


def _sc_body(g_hbm, w_hbm, i_hbm, b_hbm, out_hbm,
             g0, g1, w0, w1, i0, i1, bbuf, acc, sem0, sem1):
    wid = lax.axis_index("s") * NC + lax.axis_index("c")
    r_base = wid * R_PER_W

    zeros = jnp.zeros((LANES,), jnp.float32)

    def zbody(k, _):
        acc[pl.ds(k * LANES, LANES)] = zeros
        return _

    lax.fori_loop(0, ACC_W // LANES, zbody, None)

    pltpu.sync_copy(b_hbm, bbuf)
    bv = bbuf[pl.ds(wid * LANES, LANES)]
    s0 = bv[0]
    s1 = bv[1]
    s0a = (s0 // LANES) * LANES
    nchunks = (s1 - s0a + C - 1) // C
    lanes = lax.iota(jnp.int32, LANES)

    def start_chunk(c, gb, wb, ib, sem):
        cs = jnp.minimum(s0a + c * C, N - C)
        pltpu.make_async_copy(g_hbm.at[pl.ds(cs * D, C * D)], gb, sem).start()
        pltpu.make_async_copy(w_hbm.at[pl.ds(cs, C)], wb, sem).start()
        pltpu.make_async_copy(i_hbm.at[pl.ds(cs, C)], ib, sem).start()

    def wait_chunk(gb, wb, ib, sem):
        pltpu.make_async_copy(g_hbm.at[pl.ds(0, C * D)], gb, sem).wait()
        pltpu.make_async_copy(w_hbm.at[pl.ds(0, C)], wb, sem).wait()
        pltpu.make_async_copy(i_hbm.at[pl.ds(0, C)], ib, sem).wait()

    def compute(c, gb, wb, ib):
        nom = s0a + c * C
        cs = jnp.minimum(nom, N - C)
        vlo = jnp.maximum(s0, nom)

        def sbody(i16, _):
            base = i16 * LANES
            widx = ib[pl.ds(base, LANES)]
            wv = wb[pl.ds(base, LANES)]
            gav = (cs + base) + lanes
            vmask = jnp.logical_and(gav >= vlo, gav < s1)
            wv = jnp.where(vmask, wv, 0.0)
            obasev = jnp.clip(widx - r_base, 0, R_PER_W - 1) * D

            first = widx[0]
            last = widx[LANES - 1]
            uniform = first == last
            ids = [widx[l] for l in range(LANES)]
            wls = [wv[l] for l in range(LANES)]
            wpre = [jnp.where(ids[l] == first, wls[l], 0.0)
                    for l in range(LANES)]
            wpost = [jnp.where(ids[l] == last, wls[l], 0.0)
                     for l in range(LANES)]
            mids = [jnp.logical_and(ids[l] != first, ids[l] != last)
                    for l in range(1, LANES - 1)]
            has_mid = functools.reduce(jnp.logical_or, mids)

            zero = jnp.zeros((LANES,), jnp.float32)
            pre = [zero] * (D // LANES)
            post = [zero] * (D // LANES)
            for l in range(LANES):
                ibase = (base + l) * D
                wa = wpre[l]
                wb_ = wpost[l]
                for j in range(D // LANES):
                    row = gb[pl.ds(ibase + j * LANES, LANES)]
                    pre[j] = pre[j] + row * wa
                    post[j] = post[j] + row * wb_

            opre = obasev[0]
            opost = obasev[LANES - 1]

            def flush_uniform():
                for j in range(D // LANES):
                    plsc.addupdate(acc.at[pl.ds(opre + j * LANES, LANES)],
                                   pre[j])

            def flush_split():
                for j in range(D // LANES):
                    plsc.addupdate(acc.at[pl.ds(opre + j * LANES, LANES)],
                                   pre[j])
                    plsc.addupdate(acc.at[pl.ds(opost + j * LANES, LANES)],
                                   post[j])

                def middle():
                    for l in range(1, LANES - 1):
                        ibase = (base + l) * D
                        wm = jnp.where(mids[l - 1], wls[l], 0.0)
                        om = obasev[l]
                        for j in range(D // LANES):
                            row = gb[pl.ds(ibase + j * LANES, LANES)]
                            plsc.addupdate(
                                acc.at[pl.ds(om + j * LANES, LANES)],
                                row * wm)

                lax.cond(has_mid, middle, lambda: None)

            lax.cond(uniform, flush_uniform, flush_split)
            return _

        lax.fori_loop(0, C // LANES, sbody, None)

    start_chunk(0, g0, w0, i0, sem0)

    def outer(co, _):
        c = 2 * co
        start_chunk(c + 1, g1, w1, i1, sem1)
        wait_chunk(g0, w0, i0, sem0)
        compute(c, g0, w0, i0)
        start_chunk(c + 2, g0, w0, i0, sem0)
        wait_chunk(g1, w1, i1, sem1)
        compute(c + 1, g1, w1, i1)
        return _

    lax.fori_loop(0, (nchunks + 1) // 2, outer, None)
    wait_chunk(g0, w0, i0, sem0)

    pltpu.sync_copy(acc.at[pl.ds(0, ACC_W)], out_hbm.at[wid])


@jax.jit
def _sc_segment_sum(g_flat, w_flat, idx, bounds):
    mesh = plsc.VectorSubcoreMesh(core_axis_name="c", subcore_axis_name="s",
                                  num_cores=NC, num_subcores=NS)
    f = pl.kernel(
        _sc_body,
        out_type=jax.ShapeDtypeStruct((NW, ACC_W), jnp.float32),
        mesh=mesh,
        scratch_types=[
            pltpu.VMEM((C * D,), jnp.float32),
            pltpu.VMEM((C * D,), jnp.float32),
            pltpu.VMEM((C,), jnp.float32),
            pltpu.VMEM((C,), jnp.float32),
            pltpu.VMEM((C,), jnp.int32),
            pltpu.VMEM((C,), jnp.int32),
            pltpu.VMEM((NW * LANES,), jnp.int32),
            pltpu.VMEM((ACC_W + D,), jnp.float32),
            pltpu.SemaphoreType.DMA,
            pltpu.SemaphoreType.DMA,
        ],
    )
    return f(g_flat, w_flat, idx, bounds)


def kernel(group, weights, ray_indices, num_rays):
    del num_rays
    idx = ray_indices.astype(jnp.int32)
    qs = jnp.arange(NW + 1, dtype=jnp.int32) * R_PER_W
    b = jnp.searchsorted(idx, qs, side="left").astype(jnp.int32)
    bounds = jnp.pad(jnp.stack([b[:-1], b[1:]], axis=1),
                     ((0, 0), (0, LANES - 2))).reshape(NW * LANES)
    g_flat = group.reshape(N * D)
    w_flat = weights.reshape(N)
    res = _sc_segment_sum(g_flat, w_flat, idx, bounds)
    return res.reshape(R_PAD, D)[:R_OUT]
